# baseline (speedup 1.0000x reference)
import functools

import jax
import jax.numpy as jnp
from jax import lax
from jax.experimental import pallas as pl
from jax.experimental.pallas import tpu as pltpu
from jax.experimental.pallas import tpu_sc as plsc

_PROB = 0.5
_CHUNK = 32


def _rank_body(L, noise_ref, noise_t_ref, out_ref):
    b = pl.program_id(0)
    B = noise_ref.shape[0]
    bsel0 = lax.broadcasted_iota(jnp.int32, (B, L), 0) == b
    row = jnp.sum(jnp.where(bsel0, noise_ref[...], 0.0), axis=0,
                  keepdims=True)
    bsel1 = lax.broadcasted_iota(jnp.int32, (L, B), 1) == b
    col = jnp.sum(jnp.where(bsel1, noise_t_ref[...], 0.0), axis=1,
                  keepdims=True)
    lt = row < col
    eq = row == col
    ii = lax.broadcasted_iota(jnp.int32, (L, L), 0)
    jj = lax.broadcasted_iota(jnp.int32, (L, L), 1)
    before = lt | (eq & (jj < ii))
    cnt = jnp.sum(jnp.where(before, 1, 0), axis=1, keepdims=True)

    Pp = out_ref.shape[2]
    p = lax.broadcasted_iota(jnp.int32, (L, Pp), 1)
    match = cnt == (p - 1)
    ival = lax.broadcasted_iota(jnp.int32, (L, Pp), 0) + 1
    idx = jnp.sum(jnp.where(match, ival, 0), axis=0, keepdims=True)
    out_ref[0] = idx


def _gather_rows(xt, gidx, n_rows, D):
    mesh = plsc.VectorSubcoreMesh(core_axis_name="c", subcore_axis_name="s")
    C = _CHUNK
    n_chunks = n_rows // C
    n_workers = 32
    per_worker = (n_chunks + n_workers - 1) // n_workers
    min_cnt = n_chunks // n_workers

    @functools.partial(
        pl.kernel,
        out_type=jax.ShapeDtypeStruct((n_rows, D), xt.dtype),
        mesh=mesh,
        scratch_types=[
            pltpu.VMEM((per_worker * C,), jnp.int32),
            pltpu.VMEM((C, D), xt.dtype),
            pltpu.VMEM((C, D), xt.dtype),
            pltpu.SemaphoreType.DMA,
            pltpu.SemaphoreType.DMA,
            pltpu.SemaphoreType.DMA,
            pltpu.SemaphoreType.DMA,
        ],
    )
    def gather_kernel(x_hbm, i_hbm, o_hbm, idx_v, rows0, rows1,
                      gs0, gs1, ws0, ws1):
        wid = lax.axis_index("s") * 2 + lax.axis_index("c")
        c0 = wid * n_chunks // n_workers
        c1 = (wid + 1) * n_chunks // n_workers
        rows, gs, ws = [rows0, rows1], [gs0, gs1], [ws0, ws1]

        def g_start(j, s):
            pltpu.async_copy(x_hbm.at[idx_v.at[pl.ds(j * C, C)]],
                             rows[s], gs[s])

        def g_wait(s):
            pltpu.make_async_copy(x_hbm.at[pl.ds(0, C)], rows[s],
                                  gs[s]).wait()

        def w_start(j, s):
            pltpu.async_copy(rows[s], o_hbm.at[pl.ds((c0 + j) * C, C)],
                             ws[s])

        def w_wait(s):
            pltpu.make_async_copy(rows[s], o_hbm.at[pl.ds(0, C)],
                                  ws[s]).wait()

        pltpu.sync_copy(i_hbm.at[pl.ds(c0 * C, per_worker * C)], idx_v)
        g_start(0, 0)
        for j in range(per_worker):
            s, t = j % 2, (j + 1) % 2
            if j + 1 < per_worker:
                def pre(j=j, t=t):
                    if j >= 1:
                        w_wait(t)
                    g_start(j + 1, t)
                if j + 1 <= min_cnt - 1:
                    pre()
                else:
                    pl.when(c0 + j + 1 < c1)(pre)

            def cons(j=j, s=s):
                g_wait(s)
                w_start(j, s)
            if j <= min_cnt - 1:
                cons()
            else:
                pl.when(c0 + j < c1)(cons)
        w_wait(0)
        w_wait(1)

    return gather_kernel(xt, gidx)


def kernel(x, noise):
    B, SEQ, D = x.shape
    L = SEQ - 1
    K = max(1, int(L * (1.0 - _PROB)))
    P = 1 + K
    Pp = (P + 7) & ~7

    lidx3 = pl.pallas_call(
        functools.partial(_rank_body, L),
        grid=(B,),
        in_specs=[
            pl.BlockSpec((B, L), lambda b: (0, 0)),
            pl.BlockSpec((L, B), lambda b: (0, 0)),
        ],
        out_specs=pl.BlockSpec((1, 1, Pp), lambda b: (b, 0, 0)),
        out_shape=jax.ShapeDtypeStruct((B, 1, Pp), jnp.int32),
    )(noise, noise.T)

    lidx = lidx3[:, 0, :P]
    gidx = (lidx.T * B + jnp.arange(B, dtype=jnp.int32)[None, :]).reshape(-1)

    xt = x.transpose(1, 0, 2).reshape(SEQ * B, D)
    out_t = _gather_rows(xt, gidx, P * B, D)
    return out_t.reshape(P, B, D).transpose(1, 0, 2)


# speedup vs baseline: 1.0009x; 1.0009x over previous
import functools

import jax
import jax.numpy as jnp
from jax import lax
from jax.experimental import pallas as pl
from jax.experimental.pallas import tpu as pltpu
from jax.experimental.pallas import tpu_sc as plsc

_PROB = 0.5
_CHUNK = 64


def _rank_body(L, noise_ref, noise_t_ref, out_ref):
    b = pl.program_id(0)
    B = noise_ref.shape[0]
    bsel0 = lax.broadcasted_iota(jnp.int32, (B, L), 0) == b
    row = jnp.sum(jnp.where(bsel0, noise_ref[...], 0.0), axis=0,
                  keepdims=True)
    bsel1 = lax.broadcasted_iota(jnp.int32, (L, B), 1) == b
    col = jnp.sum(jnp.where(bsel1, noise_t_ref[...], 0.0), axis=1,
                  keepdims=True)
    lt = row < col
    eq = row == col
    ii = lax.broadcasted_iota(jnp.int32, (L, L), 0)
    jj = lax.broadcasted_iota(jnp.int32, (L, L), 1)
    before = lt | (eq & (jj < ii))
    cnt = jnp.sum(jnp.where(before, 1, 0), axis=1, keepdims=True)

    Pp = out_ref.shape[2]
    p = lax.broadcasted_iota(jnp.int32, (L, Pp), 1)
    match = cnt == (p - 1)
    ival = lax.broadcasted_iota(jnp.int32, (L, Pp), 0) + 1
    idx = jnp.sum(jnp.where(match, ival, 0), axis=0, keepdims=True)
    out_ref[0] = idx


def _gather_rows(xt, gidx, n_rows, D):
    mesh = plsc.VectorSubcoreMesh(core_axis_name="c", subcore_axis_name="s")
    C = _CHUNK
    n_chunks = n_rows // C
    n_workers = 32
    per_worker = (n_chunks + n_workers - 1) // n_workers
    min_cnt = n_chunks // n_workers

    @functools.partial(
        pl.kernel,
        out_type=jax.ShapeDtypeStruct((n_rows, D), xt.dtype),
        mesh=mesh,
        scratch_types=[
            pltpu.VMEM((per_worker * C,), jnp.int32),
            pltpu.VMEM((C, D), xt.dtype),
            pltpu.VMEM((C, D), xt.dtype),
            pltpu.SemaphoreType.DMA,
            pltpu.SemaphoreType.DMA,
            pltpu.SemaphoreType.DMA,
            pltpu.SemaphoreType.DMA,
        ],
    )
    def gather_kernel(x_hbm, i_hbm, o_hbm, idx_v, rows0, rows1,
                      gs0, gs1, ws0, ws1):
        wid = lax.axis_index("s") * 2 + lax.axis_index("c")
        c0 = wid * n_chunks // n_workers
        c1 = (wid + 1) * n_chunks // n_workers
        rows, gs, ws = [rows0, rows1], [gs0, gs1], [ws0, ws1]

        def g_start(j, s):
            pltpu.async_copy(x_hbm.at[idx_v.at[pl.ds(j * C, C)]],
                             rows[s], gs[s])

        def g_wait(s):
            pltpu.make_async_copy(x_hbm.at[pl.ds(0, C)], rows[s],
                                  gs[s]).wait()

        def w_start(j, s):
            pltpu.async_copy(rows[s], o_hbm.at[pl.ds((c0 + j) * C, C)],
                             ws[s])

        def w_wait(s):
            pltpu.make_async_copy(rows[s], o_hbm.at[pl.ds(0, C)],
                                  ws[s]).wait()

        pltpu.sync_copy(i_hbm.at[pl.ds(c0 * C, per_worker * C)], idx_v)
        g_start(0, 0)
        for j in range(per_worker):
            s, t = j % 2, (j + 1) % 2
            if j + 1 < per_worker:
                def pre(j=j, t=t):
                    if j >= 1:
                        w_wait(t)
                    g_start(j + 1, t)
                if j + 1 <= min_cnt - 1:
                    pre()
                else:
                    pl.when(c0 + j + 1 < c1)(pre)

            def cons(j=j, s=s):
                g_wait(s)
                w_start(j, s)
            if j <= min_cnt - 1:
                cons()
            else:
                pl.when(c0 + j < c1)(cons)
        w_wait(0)
        w_wait(1)

    return gather_kernel(xt, gidx)


def kernel(x, noise):
    B, SEQ, D = x.shape
    L = SEQ - 1
    K = max(1, int(L * (1.0 - _PROB)))
    P = 1 + K
    Pp = (P + 7) & ~7

    lidx3 = pl.pallas_call(
        functools.partial(_rank_body, L),
        grid=(B,),
        in_specs=[
            pl.BlockSpec((B, L), lambda b: (0, 0)),
            pl.BlockSpec((L, B), lambda b: (0, 0)),
        ],
        out_specs=pl.BlockSpec((1, 1, Pp), lambda b: (b, 0, 0)),
        out_shape=jax.ShapeDtypeStruct((B, 1, Pp), jnp.int32),
    )(noise, noise.T)

    lidx = lidx3[:, 0, :P]
    gidx = (lidx.T * B + jnp.arange(B, dtype=jnp.int32)[None, :]).reshape(-1)

    xt = x.transpose(1, 0, 2).reshape(SEQ * B, D)
    out_t = _gather_rows(xt, gidx, P * B, D)
    return out_t.reshape(P, B, D).transpose(1, 0, 2)


# speedup vs baseline: 1.0770x; 1.0760x over previous
import functools

import jax
import jax.numpy as jnp
from jax import lax
from jax.experimental import pallas as pl
from jax.experimental.pallas import tpu as pltpu
from jax.experimental.pallas import tpu_sc as plsc

_PROB = 0.5
_CHUNK = 64


def _rank_body(L, noise_ref, noise_t_ref, out_ref):
    G = out_ref.shape[0]
    q = pl.program_id(0)
    B = noise_ref.shape[0]
    Pp = out_ref.shape[2]
    ii = lax.broadcasted_iota(jnp.int32, (L, L), 0)
    jj = lax.broadcasted_iota(jnp.int32, (L, L), 1)
    jlt = jj < ii
    p = lax.broadcasted_iota(jnp.int32, (L, Pp), 1)
    ival = lax.broadcasted_iota(jnp.int32, (L, Pp), 0) + 1
    for u in range(G):
        b = q * G + u
        bsel0 = lax.broadcasted_iota(jnp.int32, (B, L), 0) == b
        row = jnp.sum(jnp.where(bsel0, noise_ref[...], 0.0), axis=0,
                      keepdims=True)
        bsel1 = lax.broadcasted_iota(jnp.int32, (L, B), 1) == b
        col = jnp.sum(jnp.where(bsel1, noise_t_ref[...], 0.0), axis=1,
                      keepdims=True)
        lt = row < col
        eq = row == col
        before = lt | (eq & jlt)
        cnt = jnp.sum(jnp.where(before, 1, 0), axis=1,
                      keepdims=True)
        match = cnt == (p - 1)
        idx = jnp.sum(jnp.where(match, ival, 0), axis=0,
                      keepdims=True)
        out_ref[u] = idx


def _gather_rows(xt, gidx, n_rows, D):
    mesh = plsc.VectorSubcoreMesh(core_axis_name="c", subcore_axis_name="s")
    C = _CHUNK
    n_chunks = n_rows // C
    n_workers = 32
    per_worker = (n_chunks + n_workers - 1) // n_workers
    min_cnt = n_chunks // n_workers

    @functools.partial(
        pl.kernel,
        out_type=jax.ShapeDtypeStruct((n_rows, D), xt.dtype),
        mesh=mesh,
        scratch_types=[
            pltpu.VMEM((per_worker * C,), jnp.int32),
            pltpu.VMEM((C, D), xt.dtype),
            pltpu.VMEM((C, D), xt.dtype),
            pltpu.SemaphoreType.DMA,
            pltpu.SemaphoreType.DMA,
            pltpu.SemaphoreType.DMA,
            pltpu.SemaphoreType.DMA,
        ],
    )
    def gather_kernel(x_hbm, i_hbm, o_hbm, idx_v, rows0, rows1,
                      gs0, gs1, ws0, ws1):
        wid = lax.axis_index("s") * 2 + lax.axis_index("c")
        c0 = wid * n_chunks // n_workers
        c1 = (wid + 1) * n_chunks // n_workers
        rows, gs, ws = [rows0, rows1], [gs0, gs1], [ws0, ws1]

        def g_start(j, s):
            pltpu.async_copy(x_hbm.at[idx_v.at[pl.ds(j * C, C)]],
                             rows[s], gs[s])

        def g_wait(s):
            pltpu.make_async_copy(x_hbm.at[pl.ds(0, C)], rows[s],
                                  gs[s]).wait()

        def w_start(j, s):
            pltpu.async_copy(rows[s], o_hbm.at[pl.ds((c0 + j) * C, C)],
                             ws[s])

        def w_wait(s):
            pltpu.make_async_copy(rows[s], o_hbm.at[pl.ds(0, C)],
                                  ws[s]).wait()

        pltpu.sync_copy(i_hbm.at[pl.ds(c0 * C, per_worker * C)], idx_v)
        g_start(0, 0)
        for j in range(per_worker):
            s, t = j % 2, (j + 1) % 2
            if j + 1 < per_worker:
                def pre(j=j, t=t):
                    if j >= 1:
                        w_wait(t)
                    g_start(j + 1, t)
                if j + 1 <= min_cnt - 1:
                    pre()
                else:
                    pl.when(c0 + j + 1 < c1)(pre)

            def cons(j=j, s=s):
                g_wait(s)
                w_start(j, s)
            if j <= min_cnt - 1:
                cons()
            else:
                pl.when(c0 + j < c1)(cons)
        w_wait(0)
        w_wait(1)

    return gather_kernel(xt, gidx)


def kernel(x, noise):
    B, SEQ, D = x.shape
    L = SEQ - 1
    K = max(1, int(L * (1.0 - _PROB)))
    P = 1 + K
    Pp = (P + 7) & ~7

    G = 8
    lidx3 = pl.pallas_call(
        functools.partial(_rank_body, L),
        grid=(B // G,),
        in_specs=[
            pl.BlockSpec((B, L), lambda q: (0, 0)),
            pl.BlockSpec((L, B), lambda q: (0, 0)),
        ],
        out_specs=pl.BlockSpec((G, 1, Pp), lambda q: (q, 0, 0)),
        out_shape=jax.ShapeDtypeStruct((B, 1, Pp), jnp.int32),
    )(noise, noise.T)

    lidx = lidx3[:, 0, :P]
    gidx = (lidx.T * B + jnp.arange(B, dtype=jnp.int32)[None, :]).reshape(-1)

    xt = x.transpose(1, 0, 2).reshape(SEQ * B, D)
    out_t = _gather_rows(xt, gidx, P * B, D)
    return out_t.reshape(P, B, D).transpose(1, 0, 2)


# speedup vs baseline: 1.0836x; 1.0061x over previous
import functools

import jax
import jax.numpy as jnp
from jax import lax
from jax.experimental import pallas as pl
from jax.experimental.pallas import tpu as pltpu
from jax.experimental.pallas import tpu_sc as plsc

_PROB = 0.5
_CHUNK = 64


def _rank_body(L, noise_ref, noise_t_ref, out_ref):
    G = out_ref.shape[0]
    q = pl.program_id(0)
    B = noise_ref.shape[0]
    Pp = out_ref.shape[2]
    ii = lax.broadcasted_iota(jnp.int32, (L, L), 0)
    jj = lax.broadcasted_iota(jnp.int32, (L, L), 1)
    jlt = jj < ii
    p = lax.broadcasted_iota(jnp.int32, (L, Pp), 1)
    ival = lax.broadcasted_iota(jnp.int32, (L, Pp), 0) + 1
    for u in range(G):
        b = q * G + u
        bsel0 = lax.broadcasted_iota(jnp.int32, (B, L), 0) == b
        row = jnp.sum(jnp.where(bsel0, noise_ref[...], 0.0), axis=0,
                      keepdims=True)
        bsel1 = lax.broadcasted_iota(jnp.int32, (L, B), 1) == b
        col = jnp.sum(jnp.where(bsel1, noise_t_ref[...], 0.0), axis=1,
                      keepdims=True)
        lt = row < col
        eq = row == col
        before = lt | (eq & jlt)
        cnt = jnp.sum(jnp.where(before, 1, 0), axis=1,
                      keepdims=True)
        match = cnt == (p - 1)
        idx = jnp.sum(jnp.where(match, ival, 0), axis=0,
                      keepdims=True)
        out_ref[u] = idx


def _gather_rows(xt, gidx, n_rows, D):
    mesh = plsc.VectorSubcoreMesh(core_axis_name="c", subcore_axis_name="s")
    C = _CHUNK
    n_chunks = n_rows // C
    n_workers = 32
    per_worker = (n_chunks + n_workers - 1) // n_workers
    min_cnt = n_chunks // n_workers

    @functools.partial(
        pl.kernel,
        out_type=jax.ShapeDtypeStruct((n_rows, D), xt.dtype),
        mesh=mesh,
        scratch_types=[
            pltpu.VMEM((per_worker * C,), jnp.int32),
            pltpu.VMEM((C, D), xt.dtype),
            pltpu.VMEM((C, D), xt.dtype),
            pltpu.SemaphoreType.DMA,
            pltpu.SemaphoreType.DMA,
            pltpu.SemaphoreType.DMA,
            pltpu.SemaphoreType.DMA,
        ],
    )
    def gather_kernel(x_hbm, i_hbm, o_hbm, idx_v, rows0, rows1,
                      gs0, gs1, ws0, ws1):
        wid = lax.axis_index("s") * 2 + lax.axis_index("c")
        c0 = wid * n_chunks // n_workers
        c1 = (wid + 1) * n_chunks // n_workers
        rows, gs, ws = [rows0, rows1], [gs0, gs1], [ws0, ws1]

        def g_start(j, s):
            pltpu.async_copy(x_hbm.at[idx_v.at[pl.ds(j * C, C)]],
                             rows[s], gs[s])

        def g_wait(s):
            pltpu.make_async_copy(x_hbm.at[pl.ds(0, C)], rows[s],
                                  gs[s]).wait()

        def w_start(j, s):
            pltpu.async_copy(rows[s], o_hbm.at[pl.ds((c0 + j) * C, C)],
                             ws[s])

        def w_wait(s):
            pltpu.make_async_copy(rows[s], o_hbm.at[pl.ds(0, C)],
                                  ws[s]).wait()

        pltpu.sync_copy(i_hbm.at[pl.ds(c0 * C, per_worker * C)], idx_v)
        g_start(0, 0)
        for j in range(per_worker):
            s, t = j % 2, (j + 1) % 2
            if j + 1 < per_worker:
                def pre(j=j, t=t):
                    if j >= 1:
                        w_wait(t)
                    g_start(j + 1, t)
                if j + 1 <= min_cnt - 1:
                    pre()
                else:
                    pl.when(c0 + j + 1 < c1)(pre)

            def cons(j=j, s=s):
                g_wait(s)
                w_start(j, s)
            if j <= min_cnt - 1:
                cons()
            else:
                pl.when(c0 + j < c1)(cons)
        w_wait(0)
        w_wait(1)

    return gather_kernel(xt, gidx)


def kernel(x, noise):
    B, SEQ, D = x.shape
    L = SEQ - 1
    K = max(1, int(L * (1.0 - _PROB)))
    P = 1 + K
    Pp = (P + 7) & ~7

    G = 16
    lidx3 = pl.pallas_call(
        functools.partial(_rank_body, L),
        grid=(B // G,),
        in_specs=[
            pl.BlockSpec((B, L), lambda q: (0, 0)),
            pl.BlockSpec((L, B), lambda q: (0, 0)),
        ],
        out_specs=pl.BlockSpec((G, 1, Pp), lambda q: (q, 0, 0)),
        out_shape=jax.ShapeDtypeStruct((B, 1, Pp), jnp.int32),
    )(noise, noise.T)

    lidx = lidx3[:, 0, :P]
    gidx = (lidx.T * B + jnp.arange(B, dtype=jnp.int32)[None, :]).reshape(-1)

    xt = x.transpose(1, 0, 2).reshape(SEQ * B, D)
    out_t = _gather_rows(xt, gidx, P * B, D)
    return out_t.reshape(P, B, D).transpose(1, 0, 2)
